# baseline (speedup 1.0000x reference)
import jax
import jax.numpy as jnp
from jax import lax
from jax.experimental import pallas as pl
from jax.experimental.pallas import tpu as pltpu
from jax.experimental.pallas import tpu_sc as plsc

_N = 10000
_E = 320000
_LANE = 128
_ROWS = 2560
_EPAD = _ROWS * _LANE
_NACC = 10240
_NC, _NS = 2, 16
_NW = _NC * _NS
_RPW = _ROWS // _NW
_RCHA = 16
_RCHB = 4
_NPT = _NACC // _NS

_f32 = jnp.float32


def _pre_node_body(x_r, w_r, b_r, a1_r, a2_r, fs_r, al_r, be_r):
    f = jnp.dot(x_r[...], w_r[...], preferred_element_type=_f32) + b_r[...]
    fs_r[...] = f
    al_r[...] = jnp.dot(f, a1_r[...], preferred_element_type=_f32)
    be_r[...] = jnp.dot(f, a2_r[...], preferred_element_type=_f32)


def _pre_edge_body(ef_r, w_r, b_r, a2_r, relu_r, gam_r):
    f = jnp.dot(ef_r[...], w_r[...], preferred_element_type=_f32) + b_r[...]
    relu_r[...] = jnp.maximum(f, 0.0)
    gam_r[...] = jnp.dot(f, a2_r[...], preferred_element_type=_f32)


def _final_body(hp_r, gp_r, s2_r, fs_r, we_r, bebias_r, wu_r, bu_r,
                wih_r, whh_r, bih_r, bhh_r, out_r):
    h1 = jnp.maximum(hp_r[0, :_N, :] + hp_r[1, :_N, :], 0.0)
    g = gp_r[0, :_N, :] + gp_r[1, :_N, :]
    s2 = s2_r[0, :_N, :] + s2_r[1, :_N, :]
    mask = (s2 > 0.0).astype(_f32)
    rinv = jnp.where(s2 > 0.0, 1.0 / s2, 0.0)
    gn = g * rinv
    e2n = jnp.dot(gn, we_r[...], preferred_element_type=_f32) + mask * bebias_r[...]
    e2n = jnp.maximum(e2n, 0.0)
    cat = jnp.concatenate([h1, e2n], axis=1)
    x = jnp.maximum(jnp.dot(cat, wu_r[...], preferred_element_type=_f32) + bu_r[...], 0.0)
    h0 = fs_r[...]
    gi = lax.dot_general(x, wih_r[...], (((1,), (1,)), ((), ())),
                         preferred_element_type=_f32) + bih_r[...]
    gh = lax.dot_general(h0, whh_r[...], (((1,), (1,)), ((), ())),
                         preferred_element_type=_f32) + bhh_r[...]
    r = jax.nn.sigmoid(gi[:, :64] + gh[:, :64])
    z = jax.nn.sigmoid(gi[:, 64:128] + gh[:, 64:128])
    n = jnp.tanh(gi[:, 128:] + r * gh[:, 128:])
    out_r[...] = (1.0 - z) * n + z * h0


def _sc_attn_body(sd_r, gam_r, alpha_r, beta_r, ef_r,
                  sparts_out, gparts_out,
                  alpha_v, beta_v, sdb, dstf, gamb, e1f, e2f, efb, zb1, zb16,
                  s1_sp, s2_sp, g_sp):
    c = lax.axis_index("c")
    s = lax.axis_index("s")
    wid = c * _NS + s
    zoff = s * _NPT
    zv = jnp.zeros((16,), _f32)
    for m in range(_NPT // 16):
        zb1[pl.ds(m * 16, 16)] = zv
    for m in range(64):
        zb16[m, :] = zv
    pltpu.sync_copy(zb1, s1_sp.at[pl.ds(zoff, _NPT)])
    pltpu.sync_copy(zb1, s2_sp.at[pl.ds(zoff, _NPT)])
    for m in range(_NPT // 64):
        pltpu.sync_copy(zb16, g_sp.at[pl.ds(zoff + m * 64, 64)])
    pltpu.sync_copy(alpha_r, alpha_v)
    pltpu.sync_copy(beta_r, beta_v)
    plsc.subcore_barrier()
    row0 = wid * _RPW

    def chunk(ci, carry):
        r0 = row0 + ci * _RCHA
        pltpu.sync_copy(sd_r.at[pl.ds(r0, _RCHA)], sdb)
        pltpu.sync_copy(gam_r.at[pl.ds(r0, _RCHA)], gamb)
        pltpu.sync_copy(ef_r.at[pl.ds(r0 * _LANE, _RCHA * _LANE)], efb)
        for j in range(_RCHA):
            for k in range(_LANE // 16):
                sl = pl.ds(k * 16, 16)
                fl = pl.ds(j * _LANE + k * 16, 16)
                sd = sdb[j, sl]
                sv = lax.bitwise_and(sd, jnp.int32(16383))
                dv = lax.shift_right_logical(sd, jnp.int32(14))
                dstf[fl] = dv
                av_s = plsc.load_gather(alpha_v, [sv])
                av_d = plsc.load_gather(alpha_v, [dv])
                bv_s = plsc.load_gather(beta_v, [sv])
                l1 = av_s + av_d
                l1 = jnp.maximum(l1, 0.2 * l1)
                l2 = bv_s + gamb[j, sl]
                l2 = jnp.maximum(l2, 0.2 * l2)
                e1f[fl] = jnp.exp(l1)
                e2f[fl] = jnp.exp(l2)

        def eblk(it, carry2):
            base = it * 16
            for l in range(16):
                i = base + l
                iv = lax.broadcast(i, (16,))
                b2 = plsc.load_gather(e2f, [iv])
                efb[i, :] = efb[i, :] * b2
            return carry2

        lax.fori_loop(0, _RCHA * _LANE // 16, eblk, 0)
        pltpu.sync_copy(e1f, s1_sp.at[dstf], add=True)
        pltpu.sync_copy(e2f, s2_sp.at[dstf], add=True)
        pltpu.sync_copy(efb, g_sp.at[dstf], add=True)
        return carry

    lax.fori_loop(0, _RPW // _RCHA, chunk, 0)
    plsc.subcore_barrier()
    off = s * _NPT
    pltpu.sync_copy(s1_sp.at[pl.ds(off, _NPT)], sparts_out.at[c, 0, pl.ds(off, _NPT)])
    pltpu.sync_copy(s2_sp.at[pl.ds(off, _NPT)], sparts_out.at[c, 1, pl.ds(off, _NPT)])
    pltpu.sync_copy(g_sp.at[pl.ds(off, _NPT)], gparts_out.at[c, pl.ds(off, _NPT)])


def _sc_agg_body(sd_r, alpha_r, sparts_r, fs_r,
                 hparts_out,
                 alpha_v, r1_v, ta, tb, sdb, srcf, dstf, a1f,
                 rows_v, zb64, h_sp, sem):
    c = lax.axis_index("c")
    s = lax.axis_index("s")
    wid = c * _NS + s
    zoff = s * _NPT

    zv = jnp.zeros((16,), _f32)
    for m in range(64):
        for q in range(4):
            zb64[m, pl.ds(q * 16, 16)] = zv
    for m in range(_NPT // 64):
        pltpu.sync_copy(zb64, h_sp.at[pl.ds(zoff + m * 64, 64)])

    pltpu.sync_copy(sparts_r.at[0, 0], ta)
    pltpu.sync_copy(sparts_r.at[1, 0], tb)

    def rec1(i, carry):
        sl = pl.ds(i * 16, 16)
        v = ta[sl] + tb[sl]
        r1_v[sl] = jnp.where(v > 0.0, 1.0 / v, 0.0)
        return carry

    lax.fori_loop(0, _NACC // 16, rec1, 0)
    pltpu.sync_copy(alpha_r, alpha_v)
    plsc.subcore_barrier()
    row0 = wid * _RPW

    def chunk(ci, carry):
        r0 = row0 + ci * _RCHB
        pltpu.sync_copy(sd_r.at[pl.ds(r0, _RCHB)], sdb)
        for j in range(_RCHB):
            for k in range(_LANE // 16):
                sl = pl.ds(k * 16, 16)
                fl0 = pl.ds(j * _LANE + k * 16, 16)
                sd = sdb[j, sl]
                srcf[fl0] = lax.bitwise_and(sd, jnp.int32(16383))
                dstf[fl0] = lax.shift_right_logical(sd, jnp.int32(14))
        desc = pltpu.async_copy(fs_r.at[srcf], rows_v, sem)
        for j in range(_RCHB):
            for k in range(_LANE // 16):
                sl = pl.ds(k * 16, 16)
                fl = pl.ds(j * _LANE + k * 16, 16)
                sv = srcf[fl]
                dv = dstf[fl]
                av_s = plsc.load_gather(alpha_v, [sv])
                av_d = plsc.load_gather(alpha_v, [dv])
                l1 = av_s + av_d
                l1 = jnp.maximum(l1, 0.2 * l1)
                a1f[fl] = jnp.exp(l1) * plsc.load_gather(r1_v, [dv])
        desc.wait()

        def eblk(it, carry2):
            base = it * 16
            for l in range(16):
                i = base + l
                iv = lax.broadcast(i, (16,))
                b1 = plsc.load_gather(a1f, [iv])
                for q in range(4):
                    qs = pl.ds(q * 16, 16)
                    rows_v[i, qs] = rows_v[i, qs] * b1
            return carry2

        lax.fori_loop(0, _RCHB * _LANE // 16, eblk, 0)
        pltpu.sync_copy(rows_v, h_sp.at[dstf], add=True)
        return carry

    lax.fori_loop(0, _RPW // _RCHB, chunk, 0)
    plsc.subcore_barrier()
    off = s * _NPT
    pltpu.sync_copy(h_sp.at[pl.ds(off, _NPT)], hparts_out.at[c, pl.ds(off, _NPT)])


_pre_node = pl.pallas_call(
    _pre_node_body,
    out_shape=(jax.ShapeDtypeStruct((_N, 64), _f32),
               jax.ShapeDtypeStruct((_N, 1), _f32),
               jax.ShapeDtypeStruct((_N, 1), _f32)),
)

_EBLK = 6400

_pre_edge = pl.pallas_call(
    _pre_edge_body,
    grid=(_E // _EBLK,),
    in_specs=[pl.BlockSpec((_EBLK, 16), lambda i: (i, 0)),
              pl.BlockSpec((16, 64), lambda i: (0, 0)),
              pl.BlockSpec((1, 64), lambda i: (0, 0)),
              pl.BlockSpec((64, 1), lambda i: (0, 0))],
    out_specs=(pl.BlockSpec((_EBLK, 64), lambda i: (i, 0)),
               pl.BlockSpec((_EBLK, 1), lambda i: (i, 0))),
    out_shape=(jax.ShapeDtypeStruct((_E, 64), _f32),
               jax.ShapeDtypeStruct((_E, 1), _f32)),
)

_final = pl.pallas_call(
    _final_body,
    out_shape=jax.ShapeDtypeStruct((_N, 64), _f32),
)

_sc_mesh = plsc.VectorSubcoreMesh(core_axis_name="c", subcore_axis_name="s")

_sc_attn = pl.kernel(
    _sc_attn_body,
    out_type=(pltpu.HBM((_NC, 2, _NACC), _f32),
              pltpu.HBM((_NC, _NACC, 16), _f32)),
    mesh=_sc_mesh,
    compiler_params=pltpu.CompilerParams(
        needs_layout_passes=False, use_tc_tiling_on_sc=False),
    scratch_types=[
        pltpu.VMEM((_NACC,), _f32),
        pltpu.VMEM((_NACC,), _f32),
        pltpu.VMEM((_RCHA, _LANE), jnp.int32),
        pltpu.VMEM((_RCHA * _LANE,), jnp.int32),
        pltpu.VMEM((_RCHA, _LANE), _f32),
        pltpu.VMEM((_RCHA * _LANE,), _f32),
        pltpu.VMEM((_RCHA * _LANE,), _f32),
        pltpu.VMEM((_RCHA * _LANE, 16), _f32),
        pltpu.VMEM((_NPT,), _f32),
        pltpu.VMEM((64, 16), _f32),
        pltpu.VMEM_SHARED((_NACC,), _f32),
        pltpu.VMEM_SHARED((_NACC,), _f32),
        pltpu.VMEM_SHARED((_NACC, 16), _f32),
    ],
)

_sc_agg = pl.kernel(
    _sc_agg_body,
    out_type=(pltpu.HBM((_NC, _NACC, 64), _f32),),
    mesh=_sc_mesh,
    compiler_params=pltpu.CompilerParams(
        needs_layout_passes=False, use_tc_tiling_on_sc=False),
    scratch_types=[
        pltpu.VMEM((_NACC,), _f32),
        pltpu.VMEM((_NACC,), _f32),
        pltpu.VMEM((_NACC,), _f32),
        pltpu.VMEM((_NACC,), _f32),
        pltpu.VMEM((_RCHB, _LANE), jnp.int32),
        pltpu.VMEM((_RCHB * _LANE,), jnp.int32),
        pltpu.VMEM((_RCHB * _LANE,), jnp.int32),
        pltpu.VMEM((_RCHB * _LANE,), _f32),
        pltpu.VMEM((_RCHB * _LANE, 64), _f32),
        pltpu.VMEM((64, 64), _f32),
        pltpu.VMEM_SHARED((_NACC, 64), _f32),
        pltpu.SemaphoreType.DMA,
    ],
)


def kernel(node_feats, edge_feats, edge_index, W_n2n, b_n2n, attn_n2n,
           W_e2n, b_e2n, attn_e2n, W_upd, b_upd, W_ih, W_hh, b_ih, b_hh):
    a1c = attn_n2n.reshape(64, 1)
    a2c = attn_e2n.reshape(64, 1)
    fs, al, be = _pre_node(node_feats, W_n2n, b_n2n.reshape(1, 64), a1c, a2c)
    new_edge, gam = _pre_edge(edge_feats, W_e2n, b_e2n.reshape(1, 64), a2c)

    pad = _EPAD - _E
    src_p = jnp.concatenate([edge_index[0], jnp.zeros((pad,), jnp.int32)])
    dpad = _N + (jnp.arange(pad, dtype=jnp.int32) % (_NACC - _N))
    dst_p = jnp.concatenate([edge_index[1], dpad])
    sd2 = (src_p + (dst_p << 14)).reshape(_ROWS, _LANE)
    gam2 = jnp.concatenate([gam.reshape(-1), jnp.zeros((pad,), _f32)]
                           ).reshape(_ROWS, _LANE)
    alp = jnp.concatenate([al.reshape(-1), jnp.zeros((_NACC - _N,), _f32)])
    bep = jnp.concatenate([be.reshape(-1), jnp.zeros((_NACC - _N,), _f32)])

    efp = jnp.concatenate([edge_feats, jnp.zeros((pad, 16), _f32)], axis=0)
    sparts, gparts = _sc_attn(sd2, gam2, alp, bep, efp)
    (hparts,) = _sc_agg(sd2, alp, sparts, fs)

    s2c = sparts[:, 1, :].reshape(_NC, _NACC, 1)
    new_node = _final(hparts, gparts, s2c, fs, W_e2n, b_e2n.reshape(1, 64),
                      W_upd, b_upd.reshape(1, 64), W_ih, W_hh,
                      b_ih.reshape(1, 192), b_hh.reshape(1, 192))
    return new_node, new_edge


# speedup vs baseline: 1.3608x; 1.3608x over previous
import jax
import jax.numpy as jnp
from jax import lax
from jax.experimental import pallas as pl
from jax.experimental.pallas import tpu as pltpu
from jax.experimental.pallas import tpu_sc as plsc

_N = 10000
_E = 320000
_NACC = 10240
_NC, _NS = 2, 16
_NW = _NC * _NS
_EPW = _E // _NW
_CHA = 2000
_CHB = 400
_NPT = _NACC // _NS

_f32 = jnp.float32


def _pre_node_body(x_r, w_r, b_r, a1_r, a2_r, fs_r, al_r, be_r):
    f = jnp.dot(x_r[...], w_r[...], preferred_element_type=_f32) + b_r[...]
    fs_r[...] = f
    al_r[...] = jnp.dot(f, a1_r[...], preferred_element_type=_f32)
    be_r[...] = jnp.dot(f, a2_r[...], preferred_element_type=_f32)


def _pre_edge_body(ef_r, w_r, b_r, a2_r, relu_r, gam_r):
    f = jnp.dot(ef_r[...], w_r[...], preferred_element_type=_f32) + b_r[...]
    relu_r[...] = jnp.maximum(f, 0.0)
    gam_r[...] = jnp.dot(f, a2_r[...], preferred_element_type=_f32)


def _final_body(hp_r, gp_r, s2_r, fs_r, we_r, bebias_r, wu_r, bu_r,
                wih_r, whh_r, bih_r, bhh_r, out_r):
    h1 = jnp.maximum(hp_r[0, :_N, :] + hp_r[1, :_N, :], 0.0)
    g = gp_r[0, :_N, :] + gp_r[1, :_N, :]
    s2 = s2_r[0, :_N, :] + s2_r[1, :_N, :]
    mask = (s2 > 0.0).astype(_f32)
    rinv = jnp.where(s2 > 0.0, 1.0 / s2, 0.0)
    gn = g * rinv
    e2n = jnp.dot(gn, we_r[...], preferred_element_type=_f32) + mask * bebias_r[...]
    e2n = jnp.maximum(e2n, 0.0)
    cat = jnp.concatenate([h1, e2n], axis=1)
    x = jnp.maximum(jnp.dot(cat, wu_r[...], preferred_element_type=_f32) + bu_r[...], 0.0)
    h0 = fs_r[...]
    gi = lax.dot_general(x, wih_r[...], (((1,), (1,)), ((), ())),
                         preferred_element_type=_f32) + bih_r[...]
    gh = lax.dot_general(h0, whh_r[...], (((1,), (1,)), ((), ())),
                         preferred_element_type=_f32) + bhh_r[...]
    r = jax.nn.sigmoid(gi[:, :64] + gh[:, :64])
    z = jax.nn.sigmoid(gi[:, 64:128] + gh[:, 64:128])
    n = jnp.tanh(gi[:, 128:] + r * gh[:, 128:])
    out_r[...] = (1.0 - z) * n + z * h0


def _sc_attn_body(sd_r, gam_r, alpha_r, beta_r, ef_r,
                  sparts_out, gparts_out,
                  alpha_v, beta_v, sdb, dstf, gamb, e1f, e2f, efb, zb1, zb16,
                  s1_sp, s2_sp, g_sp):
    c = lax.axis_index("c")
    s = lax.axis_index("s")
    wid = c * _NS + s
    zoff = s * _NPT
    zv = jnp.zeros((16,), _f32)
    for m in range(_NPT // 16):
        zb1[pl.ds(m * 16, 16)] = zv
    for m in range(64):
        zb16[m, :] = zv
    pltpu.sync_copy(zb1, s1_sp.at[pl.ds(zoff, _NPT)])
    pltpu.sync_copy(zb1, s2_sp.at[pl.ds(zoff, _NPT)])
    for m in range(_NPT // 64):
        pltpu.sync_copy(zb16, g_sp.at[pl.ds(zoff + m * 64, 64)])
    pltpu.sync_copy(alpha_r, alpha_v)
    pltpu.sync_copy(beta_r, beta_v)
    plsc.subcore_barrier()
    e00 = wid * _EPW

    def chunk(ci, carry):
        e0 = e00 + ci * _CHA
        pltpu.sync_copy(sd_r.at[pl.ds(e0, _CHA)], sdb)
        pltpu.sync_copy(gam_r.at[pl.ds(e0, _CHA)], gamb)
        pltpu.sync_copy(ef_r.at[pl.ds(e0, _CHA)], efb)
        for k in range(_CHA // 16):
            sl = pl.ds(k * 16, 16)
            sd = sdb[sl]
            sv = lax.bitwise_and(sd, jnp.int32(16383))
            dv = lax.shift_right_logical(sd, jnp.int32(14))
            dstf[sl] = dv
            av_s = plsc.load_gather(alpha_v, [sv])
            av_d = plsc.load_gather(alpha_v, [dv])
            bv_s = plsc.load_gather(beta_v, [sv])
            l1 = av_s + av_d
            l1 = jnp.maximum(l1, 0.2 * l1)
            l2 = bv_s + gamb[sl]
            l2 = jnp.maximum(l2, 0.2 * l2)
            e1f[sl] = jnp.exp(l1)
            e2f[sl] = jnp.exp(l2)

        def eblk(it, carry2):
            base = it * 16
            for l in range(16):
                i = base + l
                iv = lax.broadcast(i, (16,))
                b2 = plsc.load_gather(e2f, [iv])
                efb[i, :] = efb[i, :] * b2
            return carry2

        lax.fori_loop(0, _CHA // 16, eblk, 0)
        pltpu.sync_copy(e1f, s1_sp.at[dstf], add=True)
        pltpu.sync_copy(e2f, s2_sp.at[dstf], add=True)
        pltpu.sync_copy(efb, g_sp.at[dstf], add=True)
        return carry

    lax.fori_loop(0, _EPW // _CHA, chunk, 0)
    plsc.subcore_barrier()
    off = s * _NPT
    pltpu.sync_copy(s1_sp.at[pl.ds(off, _NPT)], sparts_out.at[c, 0, pl.ds(off, _NPT)])
    pltpu.sync_copy(s2_sp.at[pl.ds(off, _NPT)], sparts_out.at[c, 1, pl.ds(off, _NPT)])
    pltpu.sync_copy(g_sp.at[pl.ds(off, _NPT)], gparts_out.at[c, pl.ds(off, _NPT)])


def _sc_agg_body(sd_r, alpha_r, sparts_r, fs_r,
                 hparts_out,
                 alpha_v, r1_v, ta, tb, sdb, srcf, dstf, a1f,
                 rows_v, zb64, h_sp, sem):
    c = lax.axis_index("c")
    s = lax.axis_index("s")
    wid = c * _NS + s
    zoff = s * _NPT

    zv = jnp.zeros((16,), _f32)
    for m in range(64):
        for q in range(4):
            zb64[m, pl.ds(q * 16, 16)] = zv
    for m in range(_NPT // 64):
        pltpu.sync_copy(zb64, h_sp.at[pl.ds(zoff + m * 64, 64)])

    pltpu.sync_copy(sparts_r.at[0, 0], ta)
    pltpu.sync_copy(sparts_r.at[1, 0], tb)

    def rec1(i, carry):
        sl = pl.ds(i * 16, 16)
        v = ta[sl] + tb[sl]
        r1_v[sl] = jnp.where(v > 0.0, 1.0 / v, 0.0)
        return carry

    lax.fori_loop(0, _NACC // 16, rec1, 0)
    pltpu.sync_copy(alpha_r, alpha_v)
    plsc.subcore_barrier()
    e00 = wid * _EPW

    def chunk(ci, carry):
        e0 = e00 + ci * _CHB
        pltpu.sync_copy(sd_r.at[pl.ds(e0, _CHB)], sdb)
        for k in range(_CHB // 16):
            sl = pl.ds(k * 16, 16)
            sd = sdb[sl]
            srcf[sl] = lax.bitwise_and(sd, jnp.int32(16383))
            dstf[sl] = lax.shift_right_logical(sd, jnp.int32(14))
        desc = pltpu.async_copy(fs_r.at[srcf], rows_v, sem)
        for k in range(_CHB // 16):
            sl = pl.ds(k * 16, 16)
            sv = srcf[sl]
            dv = dstf[sl]
            av_s = plsc.load_gather(alpha_v, [sv])
            av_d = plsc.load_gather(alpha_v, [dv])
            l1 = av_s + av_d
            l1 = jnp.maximum(l1, 0.2 * l1)
            a1f[sl] = jnp.exp(l1) * plsc.load_gather(r1_v, [dv])
        desc.wait()

        def eblk(it, carry2):
            base = it * 16
            for l in range(16):
                i = base + l
                iv = lax.broadcast(i, (16,))
                b1 = plsc.load_gather(a1f, [iv])
                for q in range(4):
                    qs = pl.ds(q * 16, 16)
                    rows_v[i, qs] = rows_v[i, qs] * b1
            return carry2

        lax.fori_loop(0, _CHB // 16, eblk, 0)
        pltpu.sync_copy(rows_v, h_sp.at[dstf], add=True)
        return carry

    lax.fori_loop(0, _EPW // _CHB, chunk, 0)
    plsc.subcore_barrier()
    off = s * _NPT
    pltpu.sync_copy(h_sp.at[pl.ds(off, _NPT)], hparts_out.at[c, pl.ds(off, _NPT)])


_pre_node = pl.pallas_call(
    _pre_node_body,
    out_shape=(jax.ShapeDtypeStruct((_N, 64), _f32),
               jax.ShapeDtypeStruct((_N, 1), _f32),
               jax.ShapeDtypeStruct((_N, 1), _f32)),
)

_EBLK = 6400

_pre_edge = pl.pallas_call(
    _pre_edge_body,
    grid=(_E // _EBLK,),
    in_specs=[pl.BlockSpec((_EBLK, 16), lambda i: (i, 0)),
              pl.BlockSpec((16, 64), lambda i: (0, 0)),
              pl.BlockSpec((1, 64), lambda i: (0, 0)),
              pl.BlockSpec((64, 1), lambda i: (0, 0))],
    out_specs=(pl.BlockSpec((_EBLK, 64), lambda i: (i, 0)),
               pl.BlockSpec((_EBLK, 1), lambda i: (i, 0))),
    out_shape=(jax.ShapeDtypeStruct((_E, 64), _f32),
               jax.ShapeDtypeStruct((_E, 1), _f32)),
)

_final = pl.pallas_call(
    _final_body,
    out_shape=jax.ShapeDtypeStruct((_N, 64), _f32),
)

_sc_mesh = plsc.VectorSubcoreMesh(core_axis_name="c", subcore_axis_name="s")

_sc_attn = pl.kernel(
    _sc_attn_body,
    out_type=(pltpu.HBM((_NC, 2, _NACC), _f32),
              pltpu.HBM((_NC, _NACC, 16), _f32)),
    mesh=_sc_mesh,
    compiler_params=pltpu.CompilerParams(
        needs_layout_passes=False, use_tc_tiling_on_sc=False),
    scratch_types=[
        pltpu.VMEM((_NACC,), _f32),
        pltpu.VMEM((_NACC,), _f32),
        pltpu.VMEM((_CHA,), jnp.int32),
        pltpu.VMEM((_CHA,), jnp.int32),
        pltpu.VMEM((_CHA,), _f32),
        pltpu.VMEM((_CHA,), _f32),
        pltpu.VMEM((_CHA,), _f32),
        pltpu.VMEM((_CHA, 16), _f32),
        pltpu.VMEM((_NPT,), _f32),
        pltpu.VMEM((64, 16), _f32),
        pltpu.VMEM_SHARED((_NACC,), _f32),
        pltpu.VMEM_SHARED((_NACC,), _f32),
        pltpu.VMEM_SHARED((_NACC, 16), _f32),
    ],
)

_sc_agg = pl.kernel(
    _sc_agg_body,
    out_type=(pltpu.HBM((_NC, _NACC, 64), _f32),),
    mesh=_sc_mesh,
    compiler_params=pltpu.CompilerParams(
        needs_layout_passes=False, use_tc_tiling_on_sc=False),
    scratch_types=[
        pltpu.VMEM((_NACC,), _f32),
        pltpu.VMEM((_NACC,), _f32),
        pltpu.VMEM((_NACC,), _f32),
        pltpu.VMEM((_NACC,), _f32),
        pltpu.VMEM((_CHB,), jnp.int32),
        pltpu.VMEM((_CHB,), jnp.int32),
        pltpu.VMEM((_CHB,), jnp.int32),
        pltpu.VMEM((_CHB,), _f32),
        pltpu.VMEM((_CHB, 64), _f32),
        pltpu.VMEM((64, 64), _f32),
        pltpu.VMEM_SHARED((_NACC, 64), _f32),
        pltpu.SemaphoreType.DMA,
    ],
)


def kernel(node_feats, edge_feats, edge_index, W_n2n, b_n2n, attn_n2n,
           W_e2n, b_e2n, attn_e2n, W_upd, b_upd, W_ih, W_hh, b_ih, b_hh):
    a1c = attn_n2n.reshape(64, 1)
    a2c = attn_e2n.reshape(64, 1)
    fs, al, be = _pre_node(node_feats, W_n2n, b_n2n.reshape(1, 64), a1c, a2c)
    new_edge, gam = _pre_edge(edge_feats, W_e2n, b_e2n.reshape(1, 64), a2c)

    sd = edge_index[0] + (edge_index[1] << 14)
    gam1 = gam.reshape(-1)
    alp = jnp.concatenate([al.reshape(-1), jnp.zeros((_NACC - _N,), _f32)])
    bep = jnp.concatenate([be.reshape(-1), jnp.zeros((_NACC - _N,), _f32)])

    sparts, gparts = _sc_attn(sd, gam1, alp, bep, edge_feats)
    (hparts,) = _sc_agg(sd, alp, sparts, fs)

    s2c = sparts[:, 1, :].reshape(_NC, _NACC, 1)
    new_node = _final(hparts, gparts, s2c, fs, W_e2n, b_e2n.reshape(1, 64),
                      W_upd, b_upd.reshape(1, 64), W_ih, W_hh,
                      b_ih.reshape(1, 192), b_hh.reshape(1, 192))
    return new_node, new_edge


# speedup vs baseline: 1.8627x; 1.3688x over previous
import jax
import jax.numpy as jnp
from jax import lax
from jax.experimental import pallas as pl
from jax.experimental.pallas import tpu as pltpu
from jax.experimental.pallas import tpu_sc as plsc

_N = 10000
_E = 320000
_NACC = 10240
_NC, _NS = 2, 16
_NW = _NC * _NS
_EPW = _E // _NW
_CHA = 2000
_CHB = 400
_NPT = _NACC // _NS

_f32 = jnp.float32


def _pre_node_body(x_r, w_r, b_r, a1_r, a2_r, fs_r, al_r, be_r):
    f = jnp.dot(x_r[...], w_r[...], preferred_element_type=_f32) + b_r[...]
    fs_r[...] = f
    al_r[...] = jnp.dot(f, a1_r[...], preferred_element_type=_f32)
    be_r[...] = jnp.dot(f, a2_r[...], preferred_element_type=_f32)


def _pre_edge_body(eft_r, w_r, b_r, a2_r, relut_r, gam_r):
    eft = eft_r[...]
    fet = lax.dot_general(w_r[...], eft, (((0,), (0,)), ((), ())),
                          preferred_element_type=_f32) + b_r[...]
    relut_r[...] = jnp.maximum(fet, 0.0)
    g = lax.dot_general(a2_r[...], fet, (((0,), (0,)), ((), ())),
                        preferred_element_type=_f32)
    gam_r[...] = g


def _final_body(hp_r, gp_r, s2_r, fs_r, we_r, bebias_r, wu_r, bu_r,
                wih_r, whh_r, bih_r, bhh_r, out_r):
    h1 = jnp.maximum(hp_r[0, :_N, :] + hp_r[1, :_N, :], 0.0)
    g = gp_r[0, :_N, :] + gp_r[1, :_N, :]
    s2 = s2_r[0, :_N, :] + s2_r[1, :_N, :]
    mask = (s2 > 0.0).astype(_f32)
    rinv = jnp.where(s2 > 0.0, 1.0 / s2, 0.0)
    gn = g * rinv
    e2n = jnp.dot(gn, we_r[...], preferred_element_type=_f32) + mask * bebias_r[...]
    e2n = jnp.maximum(e2n, 0.0)
    cat = jnp.concatenate([h1, e2n], axis=1)
    x = jnp.maximum(jnp.dot(cat, wu_r[...], preferred_element_type=_f32) + bu_r[...], 0.0)
    h0 = fs_r[...]
    gi = lax.dot_general(x, wih_r[...], (((1,), (1,)), ((), ())),
                         preferred_element_type=_f32) + bih_r[...]
    gh = lax.dot_general(h0, whh_r[...], (((1,), (1,)), ((), ())),
                         preferred_element_type=_f32) + bhh_r[...]
    r = jax.nn.sigmoid(gi[:, :64] + gh[:, :64])
    z = jax.nn.sigmoid(gi[:, 64:128] + gh[:, 64:128])
    n = jnp.tanh(gi[:, 128:] + r * gh[:, 128:])
    out_r[...] = (1.0 - z) * n + z * h0


def _sc_attn_body(sd_r, gam_r, alpha_r, beta_r, ef_r,
                  sparts_out, gparts_out,
                  alpha_v, beta_v, sdb, dstf, gamb, e1f, e2f, efb, gsrc, zb1, zb16,
                  s1_sp, s2_sp, g_sp):
    c = lax.axis_index("c")
    s = lax.axis_index("s")
    wid = c * _NS + s
    zoff = s * _NPT
    zv = jnp.zeros((16,), _f32)
    for m in range(_NPT // 16):
        zb1[pl.ds(m * 16, 16)] = zv
    for m in range(64):
        zb16[m, :] = zv
    pltpu.sync_copy(zb1, s1_sp.at[pl.ds(zoff, _NPT)])
    pltpu.sync_copy(zb1, s2_sp.at[pl.ds(zoff, _NPT)])
    for m in range(_NPT // 64):
        pltpu.sync_copy(zb16, g_sp.at[pl.ds(zoff + m * 64, 64)])
    pltpu.sync_copy(alpha_r, alpha_v)
    pltpu.sync_copy(beta_r, beta_v)
    plsc.subcore_barrier()
    e00 = wid * _EPW

    def chunk(ci, carry):
        e0 = e00 + ci * _CHA
        pltpu.sync_copy(sd_r.at[pl.ds(e0, _CHA)], sdb)
        pltpu.sync_copy(gam_r.at[pl.ds(e0, _CHA)], gamb)
        pltpu.sync_copy(ef_r.at[pl.ds(e0, _CHA)], efb)
        for k in range(_CHA // 16):
            sl = pl.ds(k * 16, 16)
            sd = sdb[sl]
            sv = lax.bitwise_and(sd, jnp.int32(16383))
            dv = lax.shift_right_logical(sd, jnp.int32(14))
            dstf[sl] = dv
            av_s = plsc.load_gather(alpha_v, [sv])
            av_d = plsc.load_gather(alpha_v, [dv])
            bv_s = plsc.load_gather(beta_v, [sv])
            l1 = av_s + av_d
            l1 = jnp.maximum(l1, 0.2 * l1)
            l2 = bv_s + gamb[sl]
            l2 = jnp.maximum(l2, 0.2 * l2)
            e1f[sl] = jnp.exp(l1)
            e2f[sl] = jnp.exp(l2)

        def eblk(it, carry2):
            base = it * 16
            for l in range(16):
                i = base + l
                iv = lax.broadcast(i, (16,))
                b2 = plsc.load_gather(e2f, [iv])
                gsrc[i, :] = efb[i, :] * b2
            return carry2

        lax.fori_loop(0, _CHA // 16, eblk, 0)
        pltpu.sync_copy(e1f, s1_sp.at[dstf], add=True)
        pltpu.sync_copy(e2f, s2_sp.at[dstf], add=True)
        pltpu.sync_copy(gsrc, g_sp.at[dstf], add=True)
        return carry

    lax.fori_loop(0, _EPW // _CHA, chunk, 0)
    plsc.subcore_barrier()
    off = s * _NPT
    pltpu.sync_copy(s1_sp.at[pl.ds(off, _NPT)], sparts_out.at[c, 0, pl.ds(off, _NPT)])
    pltpu.sync_copy(s2_sp.at[pl.ds(off, _NPT)], sparts_out.at[c, 1, pl.ds(off, _NPT)])
    pltpu.sync_copy(g_sp.at[pl.ds(off, _NPT)], gparts_out.at[c, pl.ds(off, _NPT)])


def _sc_agg_body(sd_r, alpha_r, sparts_r, fs_r,
                 hparts_out,
                 alpha_v, r1_v, ta, tb, sdb, srcf, dstf, a1f,
                 rows_v, zb64, h_sp, sem):
    c = lax.axis_index("c")
    s = lax.axis_index("s")
    wid = c * _NS + s
    zoff = s * _NPT

    zv = jnp.zeros((16,), _f32)
    for m in range(64):
        for q in range(4):
            zb64[m, pl.ds(q * 16, 16)] = zv
    for m in range(_NPT // 64):
        pltpu.sync_copy(zb64, h_sp.at[pl.ds(zoff + m * 64, 64)])

    pltpu.sync_copy(sparts_r.at[0, 0], ta)
    pltpu.sync_copy(sparts_r.at[1, 0], tb)

    def rec1(i, carry):
        sl = pl.ds(i * 16, 16)
        v = ta[sl] + tb[sl]
        r1_v[sl] = jnp.where(v > 0.0, 1.0 / v, 0.0)
        return carry

    lax.fori_loop(0, _NACC // 16, rec1, 0)
    pltpu.sync_copy(alpha_r, alpha_v)
    plsc.subcore_barrier()
    e00 = wid * _EPW

    def chunk(ci, carry):
        e0 = e00 + ci * _CHB
        pltpu.sync_copy(sd_r.at[pl.ds(e0, _CHB)], sdb)
        for k in range(_CHB // 16):
            sl = pl.ds(k * 16, 16)
            sd = sdb[sl]
            srcf[sl] = lax.bitwise_and(sd, jnp.int32(16383))
            dstf[sl] = lax.shift_right_logical(sd, jnp.int32(14))
        desc = pltpu.async_copy(fs_r.at[srcf], rows_v, sem)
        for k in range(_CHB // 16):
            sl = pl.ds(k * 16, 16)
            sv = srcf[sl]
            dv = dstf[sl]
            av_s = plsc.load_gather(alpha_v, [sv])
            av_d = plsc.load_gather(alpha_v, [dv])
            l1 = av_s + av_d
            l1 = jnp.maximum(l1, 0.2 * l1)
            a1f[sl] = jnp.exp(l1) * plsc.load_gather(r1_v, [dv])
        desc.wait()

        def eblk(it, carry2):
            base = it * 16
            for l in range(16):
                i = base + l
                iv = lax.broadcast(i, (16,))
                b1 = plsc.load_gather(a1f, [iv])
                for q in range(4):
                    qs = pl.ds(q * 16, 16)
                    rows_v[i, qs] = rows_v[i, qs] * b1
            return carry2

        lax.fori_loop(0, _CHB // 16, eblk, 0)
        pltpu.sync_copy(rows_v, h_sp.at[dstf], add=True)
        return carry

    lax.fori_loop(0, _EPW // _CHB, chunk, 0)
    plsc.subcore_barrier()
    off = s * _NPT
    pltpu.sync_copy(h_sp.at[pl.ds(off, _NPT)], hparts_out.at[c, pl.ds(off, _NPT)])


_pre_node = pl.pallas_call(
    _pre_node_body,
    out_shape=(jax.ShapeDtypeStruct((_N, 64), _f32),
               jax.ShapeDtypeStruct((_N, 1), _f32),
               jax.ShapeDtypeStruct((_N, 1), _f32)),
)

_EBK = 32000

_pre_edge = pl.pallas_call(
    _pre_edge_body,
    grid=(_E // _EBK,),
    in_specs=[pl.BlockSpec((16, _EBK), lambda i: (0, i)),
              pl.BlockSpec((16, 64), lambda i: (0, 0)),
              pl.BlockSpec((64, 1), lambda i: (0, 0)),
              pl.BlockSpec((64, 1), lambda i: (0, 0))],
    out_specs=(pl.BlockSpec((64, _EBK), lambda i: (0, i)),
               pl.BlockSpec((1, _EBK), lambda i: (0, i))),
    out_shape=(jax.ShapeDtypeStruct((64, _E), _f32),
               jax.ShapeDtypeStruct((1, _E), _f32)),
)

_final = pl.pallas_call(
    _final_body,
    out_shape=jax.ShapeDtypeStruct((_N, 64), _f32),
)

_sc_mesh = plsc.VectorSubcoreMesh(core_axis_name="c", subcore_axis_name="s")

_sc_attn = pl.kernel(
    _sc_attn_body,
    out_type=(pltpu.HBM((_NC, 2, _NACC), _f32),
              pltpu.HBM((_NC, _NACC, 16), _f32)),
    mesh=_sc_mesh,
    compiler_params=pltpu.CompilerParams(
        needs_layout_passes=False, use_tc_tiling_on_sc=False),
    scratch_types=[
        pltpu.VMEM((_NACC,), _f32),
        pltpu.VMEM((_NACC,), _f32),
        pltpu.VMEM((_CHA,), jnp.int32),
        pltpu.VMEM((_CHA,), jnp.int32),
        pltpu.VMEM((_CHA,), _f32),
        pltpu.VMEM((_CHA,), _f32),
        pltpu.VMEM((_CHA,), _f32),
        pltpu.VMEM((_CHA, 16), _f32),
        pltpu.VMEM((_CHA, 16), _f32),
        pltpu.VMEM((_NPT,), _f32),
        pltpu.VMEM((64, 16), _f32),
        pltpu.VMEM_SHARED((_NACC,), _f32),
        pltpu.VMEM_SHARED((_NACC,), _f32),
        pltpu.VMEM_SHARED((_NACC, 16), _f32),
    ],
)

_sc_agg = pl.kernel(
    _sc_agg_body,
    out_type=(pltpu.HBM((_NC, _NACC, 64), _f32),),
    mesh=_sc_mesh,
    compiler_params=pltpu.CompilerParams(
        needs_layout_passes=False, use_tc_tiling_on_sc=False),
    scratch_types=[
        pltpu.VMEM((_NACC,), _f32),
        pltpu.VMEM((_NACC,), _f32),
        pltpu.VMEM((_NACC,), _f32),
        pltpu.VMEM((_NACC,), _f32),
        pltpu.VMEM((_CHB,), jnp.int32),
        pltpu.VMEM((_CHB,), jnp.int32),
        pltpu.VMEM((_CHB,), jnp.int32),
        pltpu.VMEM((_CHB,), _f32),
        pltpu.VMEM((_CHB, 64), _f32),
        pltpu.VMEM((64, 64), _f32),
        pltpu.VMEM_SHARED((_NACC, 64), _f32),
        pltpu.SemaphoreType.DMA,
    ],
)


def kernel(node_feats, edge_feats, edge_index, W_n2n, b_n2n, attn_n2n,
           W_e2n, b_e2n, attn_e2n, W_upd, b_upd, W_ih, W_hh, b_ih, b_hh):
    a1c = attn_n2n.reshape(64, 1)
    a2c = attn_e2n.reshape(64, 1)
    fs, al, be = _pre_node(node_feats, W_n2n, b_n2n.reshape(1, 64), a1c, a2c)
    new_edge_t, gam2 = _pre_edge(edge_feats.T, W_e2n,
                                 b_e2n.reshape(64, 1), a2c)
    new_edge = new_edge_t.T
    gam1 = gam2[0]

    sd = edge_index[0] + (edge_index[1] << 14)
    alp = jnp.concatenate([al.reshape(-1), jnp.zeros((_NACC - _N,), _f32)])
    bep = jnp.concatenate([be.reshape(-1), jnp.zeros((_NACC - _N,), _f32)])

    sparts, gparts = _sc_attn(sd, gam1, alp, bep, edge_feats)
    (hparts,) = _sc_agg(sd, alp, sparts, fs)

    s2c = sparts[:, 1, :].reshape(_NC, _NACC, 1)
    new_node = _final(hparts, gparts, s2c, fs, W_e2n, b_e2n.reshape(1, 64),
                      W_upd, b_upd.reshape(1, 64), W_ih, W_hh,
                      b_ih.reshape(1, 192), b_hh.reshape(1, 192))
    return new_node, new_edge
